# Initial kernel scaffold; baseline (speedup 1.0000x reference)
import jax
import jax.numpy as jnp
from jax.experimental import pallas as pl


def kernel(index_sym, index_skew, factor_skew, theta_sym, theta_skew_sym):
    raise NotImplementedError("write your pallas kernel here")


# speedup vs baseline: 175.7282x; 175.7282x over previous
import jax
import jax.numpy as jnp
from jax import lax
from jax.experimental import pallas as pl
from jax.experimental.pallas import tpu as pltpu
from jax.experimental.pallas import tpu_sc as plsc

D = 4096
TOT = D * D
NW = 32
PER_W = TOT // NW
CHUNK = 4096
N_CHUNKS = PER_W // CHUNK
THETA_PAD = 1048576
PACKED_N = 524288


def _sc_body(idxs_h, idxk_h, fac_h, theta_h, packed_h, realo_h, imago_h,
             idxs_v, idxk_v, wi_v, fac_v, gr_v, gw_v, io_v,
             theta_s, packed_s, sem):
    cid = lax.axis_index("c")
    sid = lax.axis_index("s")
    wid = sid * 2 + cid

    t_per = THETA_PAD // 16
    p_per = PACKED_N // 16
    pltpu.sync_copy(theta_h.at[pl.ds(sid * t_per, t_per)],
                    theta_s.at[pl.ds(sid * t_per, t_per)])
    pltpu.sync_copy(packed_h.at[pl.ds(sid * p_per, p_per)],
                    packed_s.at[pl.ds(sid * p_per, p_per)])
    plsc.subcore_barrier()

    base = wid * PER_W

    def chunk(i, carry):
        e0 = base + i * CHUNK
        pltpu.sync_copy(idxs_h.at[pl.ds(e0, CHUNK)], idxs_v)
        pltpu.sync_copy(idxk_h.at[pl.ds(e0, CHUNK)], idxk_v)
        pltpu.sync_copy(fac_h.at[pl.ds(e0, CHUNK)], fac_v)

        def wgrp(g, c1):
            col = g * 16
            v = idxk_v[pl.ds(col, 16)]
            wi_v[pl.ds(col, 16)] = lax.shift_right_logical(v, 1)
            return c1
        lax.fori_loop(0, CHUNK // 16, wgrp, 0)

        pltpu.async_copy(theta_s.at[idxs_v], gr_v, sem).wait()
        pltpu.async_copy(packed_s.at[wi_v], gw_v, sem).wait()

        def cgrp(g, c1):
            col = g * 16
            w = gw_v[pl.ds(col, 16)]
            idx = idxk_v[pl.ds(col, 16)]
            h = lax.bitwise_and(idx, 1)
            lo = lax.shift_left(w, 16)
            hi = lax.bitwise_and(w, jnp.int32(-65536))
            bits = jnp.where(h == 1, hi, lo)
            f = lax.bitcast_convert_type(bits, jnp.float32)
            io_v[pl.ds(col, 16)] = f * fac_v[pl.ds(col, 16)]
            return c1
        lax.fori_loop(0, CHUNK // 16, cgrp, 0)

        pltpu.sync_copy(gr_v, realo_h.at[pl.ds(e0, CHUNK)])
        pltpu.sync_copy(io_v, imago_h.at[pl.ds(e0, CHUNK)])
        return carry

    lax.fori_loop(0, N_CHUNKS, chunk, 0)


def kernel(index_sym, index_skew, factor_skew, theta_sym, theta_skew_sym):
    idxs1 = index_sym.reshape(TOT)
    idxk1 = index_skew.reshape(TOT)
    fac1 = factor_skew.reshape(TOT)
    theta_pad = jnp.concatenate(
        [theta_sym, jnp.zeros((THETA_PAD - theta_sym.shape[0],), jnp.float32)])
    tmp1 = jnp.concatenate(
        [jnp.zeros((1,), jnp.float32), theta_skew_sym,
         jnp.zeros((2 * PACKED_N - 1 - theta_skew_sym.shape[0],), jnp.float32)])
    packed = lax.bitcast_convert_type(
        tmp1.astype(jnp.bfloat16).reshape(PACKED_N, 2), jnp.int32)

    mesh = plsc.VectorSubcoreMesh(core_axis_name="c", subcore_axis_name="s")
    f = pl.kernel(
        _sc_body,
        out_type=(jax.ShapeDtypeStruct((TOT,), jnp.float32),
                  jax.ShapeDtypeStruct((TOT,), jnp.float32)),
        mesh=mesh,
        scratch_types=[
            pltpu.VMEM((CHUNK,), jnp.int32),
            pltpu.VMEM((CHUNK,), jnp.int32),
            pltpu.VMEM((CHUNK,), jnp.int32),
            pltpu.VMEM((CHUNK,), jnp.float32),
            pltpu.VMEM((CHUNK,), jnp.float32),
            pltpu.VMEM((CHUNK,), jnp.int32),
            pltpu.VMEM((CHUNK,), jnp.float32),
            pltpu.VMEM_SHARED((THETA_PAD,), jnp.float32),
            pltpu.VMEM_SHARED((PACKED_N,), jnp.int32),
            pltpu.SemaphoreType.DMA,
        ],
    )
    real, imag = f(idxs1, idxk1, fac1, theta_pad, packed)
    return lax.complex(real.reshape(D, D), imag.reshape(D, D))


# speedup vs baseline: 219.4286x; 1.2487x over previous
import jax
import jax.numpy as jnp
from jax import lax
from jax.experimental import pallas as pl
from jax.experimental.pallas import tpu as pltpu
from jax.experimental.pallas import tpu_sc as plsc

D = 4096
NW = 32
ROWS_PER_W = D // NW
CHUNK = D
THETA_PAD = 1048576
HALF = 524288
GRPS = CHUNK // 16


def _sc_body(idxs_h, idxk_h, fac_h, theta_h, packed_h, realo_h, imago_h,
             idxs_v, idxk_v, wi_v, fac_v, gr_v, gw_v, io_v,
             theta_s, packed_s, sem):
    cid = lax.axis_index("c")
    sid = lax.axis_index("s")
    wid = sid * 2 + cid

    t_per = THETA_PAD // 16
    p_per = HALF // 16
    pltpu.sync_copy(theta_h.at[pl.ds(sid * t_per, t_per)],
                    theta_s.at[pl.ds(sid * t_per, t_per)])
    pltpu.sync_copy(packed_h.at[pl.ds(sid * p_per, p_per)],
                    packed_s.at[pl.ds(sid * p_per, p_per)])
    plsc.subcore_barrier()

    row0 = wid * ROWS_PER_W

    def chunk(i, carry):
        r = row0 + i
        pltpu.sync_copy(idxs_h.at[r], idxs_v)
        pltpu.sync_copy(idxk_h.at[r], idxk_v)
        pltpu.sync_copy(fac_h.at[r], fac_v)

        def wgrp(g, c1):
            col = g * 16
            t = idxk_v[pl.ds(col, 16)]
            hi = t >= HALF
            wi_v[pl.ds(col, 16)] = t - jnp.where(hi, HALF, 0)
            return c1
        lax.fori_loop(0, GRPS, wgrp, 0)

        pltpu.async_copy(theta_s.at[idxs_v], gr_v, sem).wait()
        pltpu.async_copy(packed_s.at[wi_v], gw_v, sem).wait()

        def cgrp(g, c1):
            col = g * 16
            w = gw_v[pl.ds(col, 16)]
            t = idxk_v[pl.ds(col, 16)]
            hi = t >= HALF
            lob = lax.shift_left(w, 16)
            hib = lax.bitwise_and(w, jnp.int32(-65536))
            bits = jnp.where(hi, hib, lob)
            f = lax.bitcast_convert_type(bits, jnp.float32)
            io_v[pl.ds(col, 16)] = f * fac_v[pl.ds(col, 16)]
            return c1
        lax.fori_loop(0, GRPS, cgrp, 0)

        pltpu.sync_copy(gr_v, realo_h.at[r])
        pltpu.sync_copy(io_v, imago_h.at[r])
        return carry

    lax.fori_loop(0, N_CHUNKS_PER_W, chunk, 0)


N_CHUNKS_PER_W = ROWS_PER_W


def _pack_tables(theta_sym, theta_skew_sym):
    theta_pad = jnp.concatenate(
        [theta_sym, jnp.zeros((THETA_PAD - theta_sym.shape[0],), jnp.float32)])
    tmp1 = jnp.concatenate(
        [jnp.zeros((1,), jnp.float32), theta_skew_sym,
         jnp.zeros((2 * HALF - 1 - theta_skew_sym.shape[0],), jnp.float32)])
    u = lax.bitcast_convert_type(tmp1, jnp.uint32)
    one = jnp.uint32(1)
    sixteen = jnp.uint32(16)
    rb = (u + jnp.uint32(0x7FFF) +
          (lax.shift_right_logical(u, sixteen) & one))
    rb = lax.shift_right_logical(rb, sixteen)
    packed = rb[:HALF] | lax.shift_left(rb[HALF:], sixteen)
    return theta_pad, lax.bitcast_convert_type(packed, jnp.int32)


def kernel(index_sym, index_skew, factor_skew, theta_sym, theta_skew_sym):
    theta_pad, packed = _pack_tables(theta_sym, theta_skew_sym)

    mesh = plsc.VectorSubcoreMesh(core_axis_name="c", subcore_axis_name="s")
    f = pl.kernel(
        _sc_body,
        out_type=(jax.ShapeDtypeStruct((D, D), jnp.float32),
                  jax.ShapeDtypeStruct((D, D), jnp.float32)),
        mesh=mesh,
        scratch_types=[
            pltpu.VMEM((CHUNK,), jnp.int32),
            pltpu.VMEM((CHUNK,), jnp.int32),
            pltpu.VMEM((CHUNK,), jnp.int32),
            pltpu.VMEM((CHUNK,), jnp.float32),
            pltpu.VMEM((CHUNK,), jnp.float32),
            pltpu.VMEM((CHUNK,), jnp.int32),
            pltpu.VMEM((CHUNK,), jnp.float32),
            pltpu.VMEM_SHARED((THETA_PAD,), jnp.float32),
            pltpu.VMEM_SHARED((HALF,), jnp.int32),
            pltpu.SemaphoreType.DMA,
        ],
    )
    real, imag = f(index_sym, index_skew, factor_skew, theta_pad, packed)
    return lax.complex(real, imag)
